# baseline (speedup 1.0000x reference)
import functools

import jax
import jax.numpy as jnp
from jax import lax
from jax.experimental import pallas as pl
from jax.experimental.pallas import tpu as pltpu
from jax.experimental.pallas import tpu_sc as plsc

_NC = 2
_NS = 16
_NW = _NC * _NS
_K = 128
_NBUF = 2
_IR = 4
_DEGW = 16


def _sc_aggregate(N, C, EPAD, with_deg):
    TW = EPAD // _NW // _K
    NPAD = -(-(N + 1) // _K) * _K
    TZ = NPAD // _K
    ZPT = -(-TZ // _NS)
    RPT = NPAD // _NS
    assert RPT % 8 == 0 and NPAD % _NS == 0 and TW % _IR == 0

    mesh = plsc.VectorSubcoreMesh(core_axis_name="c", subcore_axis_name="s",
                                  num_cores=_NC, num_subcores=_NS)

    out_type = [jax.ShapeDtypeStruct((_NC, NPAD, C), jnp.float32)]
    scratch = (
        [pltpu.VMEM((_IR, _K), jnp.int32),
         pltpu.VMEM((_IR, _K), jnp.int32)]
        + [pltpu.VMEM((_K, C), jnp.float32) for _ in range(_NBUF)]
        + [pltpu.VMEM((_K, _DEGW), jnp.float32),
           pltpu.VMEM((_K, _DEGW), jnp.float32),
           pltpu.VMEM_SHARED((NPAD, C), jnp.float32)]
        + [pltpu.SemaphoreType.DMA for _ in range(2 * _NBUF + _IR)]
    )
    if with_deg:
        out_type.append(jax.ShapeDtypeStruct((_NC, NPAD, _DEGW), jnp.float32))
        scratch += ([pltpu.VMEM_SHARED((NPAD, _DEGW), jnp.float32)]
                    + [pltpu.SemaphoreType.DMA for _ in range(_NBUF)])

    @functools.partial(
        pl.kernel,
        out_type=tuple(out_type),
        mesh=mesh,
        scratch_types=scratch,
        compiler_params=pltpu.CompilerParams(use_tc_tiling_on_sc=False),
    )
    def agg(x_hbm, cols_hbm, rows_hbm, sum_hbm, *rest):
        if with_deg:
            deg_hbm = rest[0]
            rest = rest[1:]
        idxc_r, idxr_r = rest[0], rest[1]
        rowbufs = rest[2:2 + _NBUF]
        ones_v, zeros16_v, acc_sh = rest[2 + _NBUF:5 + _NBUF]
        gsems = rest[5 + _NBUF:5 + 2 * _NBUF]
        ssems = rest[5 + 2 * _NBUF:5 + 3 * _NBUF]
        isems = rest[5 + 3 * _NBUF:5 + 3 * _NBUF + _IR]
        if with_deg:
            deg_sh = rest[5 + 3 * _NBUF + _IR]
            dsems = rest[6 + 3 * _NBUF + _IR:6 + 4 * _NBUF + _IR]

        c = lax.axis_index("c")
        s = lax.axis_index("s")
        wbase = ((1 - c) * _NS + s) * TW

        zeros = jnp.zeros((16,), jnp.float32)
        ones = jnp.ones((16,), jnp.float32)

        def memset_row(i, carry):
            for j in range(C // 16):
                rowbufs[0][i, pl.ds(16 * j, 16)] = zeros
            ones_v[i, pl.ds(0, 16)] = ones
            zeros16_v[i, pl.ds(0, 16)] = zeros
            return carry
        lax.fori_loop(0, _K, memset_row, 0)

        def zero_chunk(k, carry):
            g = k * _NS + s
            @pl.when(g < TZ)
            def _():
                r0 = g * _K
                pltpu.sync_copy(rowbufs[0], acc_sh.at[pl.ds(r0, _K)])
                if with_deg:
                    pltpu.sync_copy(zeros16_v, deg_sh.at[pl.ds(r0, _K)])
            return carry
        lax.fori_loop(0, ZPT, zero_chunk, 0)

        plsc.subcore_barrier()

        def prefetch(g, r):
            pltpu.async_copy(cols_hbm.at[wbase + g], idxc_r.at[r], isems[r])
            pltpu.async_copy(rows_hbm.at[wbase + g], idxr_r.at[r], isems[r])

        def idx_wait(g, r):
            pltpu.make_async_copy(cols_hbm.at[wbase + g], idxc_r.at[r],
                                  isems[r]).wait()
            pltpu.make_async_copy(rows_hbm.at[wbase + g], idxr_r.at[r],
                                  isems[r]).wait()

        def gather(r, b):
            pltpu.async_copy(x_hbm.at[idxc_r.at[r]], rowbufs[b], gsems[b])

        def gather_wait(r, b):
            pltpu.make_async_copy(x_hbm.at[idxc_r.at[r]], rowbufs[b],
                                  gsems[b]).wait()

        def scatter(r, b):
            pltpu.async_copy(rowbufs[b], acc_sh.at[idxr_r.at[r]],
                             ssems[b], add=True)
            if with_deg:
                pltpu.async_copy(ones_v, deg_sh.at[idxr_r.at[r]],
                                 dsems[b], add=True)

        def scatter_wait(r, b):
            pltpu.make_async_copy(rowbufs[b], acc_sh.at[idxr_r.at[r]],
                                  ssems[b]).wait()
            if with_deg:
                pltpu.make_async_copy(ones_v, deg_sh.at[idxr_r.at[r]],
                                      dsems[b]).wait()

        prefetch(0, 0)
        prefetch(1, 1)

        def pipe(u, carry):
            for r in range(_IR):
                g = u * _IR + r
                b = r % _NBUF

                @pl.when(g >= _NBUF)
                def _():
                    scatter_wait((r - _NBUF) % _IR, b)

                idx_wait(g, r)
                gather(r, b)

                @pl.when(g + 2 < TW)
                def _():
                    prefetch(g + 2, (r + 2) % _IR)

                @pl.when(g >= 1)
                def _():
                    gather_wait((r - 1) % _IR, 1 - b)
                    scatter((r - 1) % _IR, 1 - b)
            return carry
        lax.fori_loop(0, TW // _IR, pipe, 0)

        gather_wait((TW - 1) % _IR, (TW - 1) % _NBUF)
        scatter((TW - 1) % _IR, (TW - 1) % _NBUF)
        scatter_wait((TW - 2) % _IR, (TW - 2) % _NBUF)
        scatter_wait((TW - 1) % _IR, (TW - 1) % _NBUF)

        plsc.subcore_barrier()

        r0 = s * RPT
        pltpu.sync_copy(acc_sh.at[pl.ds(r0, RPT)], sum_hbm.at[c, pl.ds(r0, RPT)])
        if with_deg:
            pltpu.sync_copy(deg_sh.at[pl.ds(r0, RPT)],
                            deg_hbm.at[c, pl.ds(r0, RPT)])

    return agg


def _dense_layer(N, C, H, relu, BN=2000):
    def body(x_ref, p0_ref, p1_ref, d0_ref, d1_ref, w_ref, b_ref, o_ref):
        inv = 1.0 / jnp.maximum(d0_ref[0, :, 0:1] + d1_ref[0, :, 0:1], 1.0)
        agg = (p0_ref[0] + p1_ref[0]) * inv
        y = b_ref[...]
        y = y + lax.dot_general(x_ref[...], w_ref[:, :C],
                                (((1,), (1,)), ((), ())),
                                preferred_element_type=jnp.float32)
        y = y + lax.dot_general(agg, w_ref[:, C:],
                                (((1,), (1,)), ((), ())),
                                preferred_element_type=jnp.float32)
        if relu:
            y = jnp.maximum(y, 0.0)
        o_ref[...] = y

    return pl.pallas_call(
        body,
        grid=(N // BN,),
        in_specs=[
            pl.BlockSpec((BN, C), lambda i: (i, 0)),
            pl.BlockSpec((1, BN, C), lambda i: (0, i, 0)),
            pl.BlockSpec((1, BN, C), lambda i: (1, i, 0)),
            pl.BlockSpec((1, BN, _DEGW), lambda i: (0, i, 0)),
            pl.BlockSpec((1, BN, _DEGW), lambda i: (1, i, 0)),
            pl.BlockSpec((H, 2 * C), lambda i: (0, 0)),
            pl.BlockSpec((1, H), lambda i: (0, 0)),
        ],
        out_specs=pl.BlockSpec((BN, H), lambda i: (i, 0)),
        out_shape=jax.ShapeDtypeStruct((N, H), jnp.float32),
    )


def kernel(x, edge_index, W1, b1, W2, b2):
    N, C = x.shape
    H = W1.shape[0]
    O = W2.shape[0]
    E = edge_index.shape[1]
    align = _NW * _K * _IR
    EPAD = -(-E // align) * align

    rows = edge_index[0]
    cols = edge_index[1]
    pad = EPAD - E
    if pad:
        NPAD = -(-(N + 1) // _K) * _K
        trash = N + jnp.arange(pad, dtype=jnp.int32) % (NPAD - N)
        rows = jnp.concatenate([rows, trash])
        cols = jnp.concatenate([cols, jnp.zeros((pad,), jnp.int32)])
    rows = rows.reshape(EPAD // _K, _K)
    cols = cols.reshape(EPAD // _K, _K)

    sums, degs = _sc_aggregate(N, C, EPAD, True)(x, cols, rows)
    h = _dense_layer(N, C, H, True)(
        x, sums, sums, degs, degs, W1, b1.reshape(1, H))

    (sums2,) = _sc_aggregate(N, H, EPAD, False)(h, cols, rows)
    out = _dense_layer(N, H, O, False)(
        h, sums2, sums2, degs, degs, W2, b2.reshape(1, O))
    return out


# speedup vs baseline: 3.9042x; 3.9042x over previous
import functools

import jax
import jax.numpy as jnp
from jax import lax
from jax.experimental import pallas as pl
from jax.experimental.pallas import tpu as pltpu
from jax.experimental.pallas import tpu_sc as plsc

_NC = 2
_NS = 16
_NW = _NC * _NS
_K = 128
_NBUF = 2
_IR = 4
_DEGW = 16


def _sc_aggregate(N, C, EPAD, with_deg):
    TW = EPAD // _NW // _K
    NPAD = -(-(N + 1) // _K) * _K
    TZ = NPAD // _K
    ZPT = -(-TZ // _NS)
    RPT = NPAD // _NS
    assert RPT % 8 == 0 and NPAD % _NS == 0 and TW % _IR == 0

    mesh = plsc.VectorSubcoreMesh(core_axis_name="c", subcore_axis_name="s",
                                  num_cores=_NC, num_subcores=_NS)

    out_type = [jax.ShapeDtypeStruct((_NC, NPAD, C), jnp.float32)]
    scratch = (
        [pltpu.VMEM((_IR, _K), jnp.int32),
         pltpu.VMEM((_IR, _K), jnp.int32)]
        + [pltpu.VMEM((_K, C), jnp.float32) for _ in range(_NBUF)]
        + [pltpu.VMEM((_K, _DEGW), jnp.float32),
           pltpu.VMEM((_K, _DEGW), jnp.float32),
           pltpu.VMEM_SHARED((NPAD, C), jnp.float32)]
        + [pltpu.SemaphoreType.DMA for _ in range(2 * _NBUF + _IR)]
    )
    if with_deg:
        out_type.append(jax.ShapeDtypeStruct((_NC, NPAD, _DEGW), jnp.float32))
        scratch += ([pltpu.VMEM_SHARED((NPAD, _DEGW), jnp.float32)]
                    + [pltpu.SemaphoreType.DMA for _ in range(_NBUF)])

    @functools.partial(
        pl.kernel,
        out_type=tuple(out_type),
        mesh=mesh,
        scratch_types=scratch,
        compiler_params=pltpu.CompilerParams(use_tc_tiling_on_sc=False),
    )
    def agg(x_hbm, cols_hbm, rows_hbm, sum_hbm, *rest):
        if with_deg:
            deg_hbm = rest[0]
            rest = rest[1:]
        idxc_r, idxr_r = rest[0], rest[1]
        rowbufs = rest[2:2 + _NBUF]
        ones_v, zeros16_v, acc_sh = rest[2 + _NBUF:5 + _NBUF]
        gsems = rest[5 + _NBUF:5 + 2 * _NBUF]
        ssems = rest[5 + 2 * _NBUF:5 + 3 * _NBUF]
        isems = rest[5 + 3 * _NBUF:5 + 3 * _NBUF + _IR]
        if with_deg:
            deg_sh = rest[5 + 3 * _NBUF + _IR]
            dsems = rest[6 + 3 * _NBUF + _IR:6 + 4 * _NBUF + _IR]

        c = lax.axis_index("c")
        s = lax.axis_index("s")
        wbase = (c * _NS + s) * TW

        zeros = jnp.zeros((16,), jnp.float32)
        ones = jnp.ones((16,), jnp.float32)

        def memset_row(i, carry):
            for j in range(C // 16):
                rowbufs[0][i, pl.ds(16 * j, 16)] = zeros
            ones_v[i, pl.ds(0, 16)] = ones
            zeros16_v[i, pl.ds(0, 16)] = zeros
            return carry
        lax.fori_loop(0, _K, memset_row, 0)

        def zero_chunk(k, carry):
            g = k * _NS + s
            @pl.when(g < TZ)
            def _():
                r0 = g * _K
                pltpu.sync_copy(rowbufs[0], acc_sh.at[pl.ds(r0, _K)])
                if with_deg:
                    pltpu.sync_copy(zeros16_v, deg_sh.at[pl.ds(r0, _K)])
            return carry
        lax.fori_loop(0, ZPT, zero_chunk, 0)

        plsc.subcore_barrier()

        def prefetch(g, r):
            pltpu.async_copy(cols_hbm.at[wbase + g], idxc_r.at[r], isems[r])
            pltpu.async_copy(rows_hbm.at[wbase + g], idxr_r.at[r], isems[r])

        def idx_wait(g, r):
            pltpu.make_async_copy(cols_hbm.at[wbase + g], idxc_r.at[r],
                                  isems[r]).wait()
            pltpu.make_async_copy(rows_hbm.at[wbase + g], idxr_r.at[r],
                                  isems[r]).wait()

        def gather(r, b):
            pltpu.async_copy(x_hbm.at[idxc_r.at[r]], rowbufs[b], gsems[b])

        def gather_wait(r, b):
            pltpu.make_async_copy(x_hbm.at[idxc_r.at[r]], rowbufs[b],
                                  gsems[b]).wait()

        def scatter(r, b):
            pltpu.async_copy(rowbufs[b], acc_sh.at[idxr_r.at[r]],
                             ssems[b], add=True)
            if with_deg:
                pltpu.async_copy(ones_v, deg_sh.at[idxr_r.at[r]],
                                 dsems[b], add=True)

        def scatter_wait(r, b):
            pltpu.make_async_copy(rowbufs[b], acc_sh.at[idxr_r.at[r]],
                                  ssems[b]).wait()
            if with_deg:
                pltpu.make_async_copy(ones_v, deg_sh.at[idxr_r.at[r]],
                                      dsems[b]).wait()

        prefetch(0, 0)
        prefetch(1, 1)

        def pipe(u, carry):
            for r in range(_IR):
                g = u * _IR + r
                b = r % _NBUF

                @pl.when(g >= _NBUF)
                def _():
                    scatter_wait((r - _NBUF) % _IR, b)

                idx_wait(g, r)
                gather(r, b)

                @pl.when(g + 2 < TW)
                def _():
                    prefetch(g + 2, (r + 2) % _IR)

                @pl.when(g >= 1)
                def _():
                    gather_wait((r - 1) % _IR, 1 - b)
                    scatter((r - 1) % _IR, 1 - b)
            return carry
        lax.fori_loop(0, TW // _IR, pipe, 0)

        gather_wait((TW - 1) % _IR, (TW - 1) % _NBUF)
        scatter((TW - 1) % _IR, (TW - 1) % _NBUF)
        scatter_wait((TW - 2) % _IR, (TW - 2) % _NBUF)
        scatter_wait((TW - 1) % _IR, (TW - 1) % _NBUF)

        plsc.subcore_barrier()

        r0 = s * RPT
        pltpu.sync_copy(acc_sh.at[pl.ds(r0, RPT)], sum_hbm.at[c, pl.ds(r0, RPT)])
        if with_deg:
            pltpu.sync_copy(deg_sh.at[pl.ds(r0, RPT)],
                            deg_hbm.at[c, pl.ds(r0, RPT)])

    return agg


def _dense_layer(N, C, H, relu, BN=2000):
    def body(x_ref, p0_ref, p1_ref, d0_ref, d1_ref, w_ref, b_ref, o_ref):
        inv = 1.0 / jnp.maximum(d0_ref[0, :, 0:1] + d1_ref[0, :, 0:1], 1.0)
        agg = (p0_ref[0] + p1_ref[0]) * inv
        y = b_ref[...]
        y = y + lax.dot_general(x_ref[...], w_ref[:, :C],
                                (((1,), (1,)), ((), ())),
                                preferred_element_type=jnp.float32)
        y = y + lax.dot_general(agg, w_ref[:, C:],
                                (((1,), (1,)), ((), ())),
                                preferred_element_type=jnp.float32)
        if relu:
            y = jnp.maximum(y, 0.0)
        o_ref[...] = y

    return pl.pallas_call(
        body,
        grid=(N // BN,),
        in_specs=[
            pl.BlockSpec((BN, C), lambda i: (i, 0)),
            pl.BlockSpec((1, BN, C), lambda i: (0, i, 0)),
            pl.BlockSpec((1, BN, C), lambda i: (1, i, 0)),
            pl.BlockSpec((1, BN, _DEGW), lambda i: (0, i, 0)),
            pl.BlockSpec((1, BN, _DEGW), lambda i: (1, i, 0)),
            pl.BlockSpec((H, 2 * C), lambda i: (0, 0)),
            pl.BlockSpec((1, H), lambda i: (0, 0)),
        ],
        out_specs=pl.BlockSpec((BN, H), lambda i: (i, 0)),
        out_shape=jax.ShapeDtypeStruct((N, H), jnp.float32),
    )


def kernel(x, edge_index, W1, b1, W2, b2):
    N, C = x.shape
    H = W1.shape[0]
    O = W2.shape[0]
    E = edge_index.shape[1]
    align = _NW * _K * _IR
    EPAD = -(-E // align) * align

    rows = edge_index[0]
    cols = edge_index[1]
    pad = EPAD - E
    if pad:
        NPAD = -(-(N + 1) // _K) * _K
        arp = jnp.arange(pad, dtype=jnp.int32)
        rows = jnp.concatenate([rows, N + arp % (NPAD - N)])
        cols = jnp.concatenate([cols, arp % N])
    rows = rows.reshape(EPAD // _K, _K)
    cols = cols.reshape(EPAD // _K, _K)

    sums, degs = _sc_aggregate(N, C, EPAD, True)(x, cols, rows)
    h = _dense_layer(N, C, H, True)(
        x, sums, sums, degs, degs, W1, b1.reshape(1, H))

    (sums2,) = _sc_aggregate(N, H, EPAD, False)(h, cols, rows)
    out = _dense_layer(N, H, O, False)(
        h, sums2, sums2, degs, degs, W2, b2.reshape(1, O))
    return out


# speedup vs baseline: 3.9099x; 1.0015x over previous
import functools

import jax
import jax.numpy as jnp
from jax import lax
from jax.experimental import pallas as pl
from jax.experimental.pallas import tpu as pltpu
from jax.experimental.pallas import tpu_sc as plsc

_NC = 2
_NS = 16
_NW = _NC * _NS
_K = 128
_NBUF = 2
_IR = 4
_DEGW = 16


def _sc_aggregate(N, C, EPAD, with_deg):
    TW = EPAD // _NW // _K
    NPAD = -(-(N + 1) // _K) * _K
    TZ = NPAD // _K
    ZPT = -(-TZ // _NS)
    RPT = NPAD // _NS
    assert RPT % 8 == 0 and NPAD % _NS == 0 and TW % _IR == 0

    mesh = plsc.VectorSubcoreMesh(core_axis_name="c", subcore_axis_name="s",
                                  num_cores=_NC, num_subcores=_NS)

    out_type = [jax.ShapeDtypeStruct((_NC, NPAD, C), jnp.float32)]
    scratch = (
        [pltpu.VMEM((_IR, 2, _K), jnp.int32)]
        + [pltpu.VMEM((_K, C), jnp.float32) for _ in range(_NBUF)]
        + [pltpu.VMEM((_K, _DEGW), jnp.float32),
           pltpu.VMEM((_K, _DEGW), jnp.float32),
           pltpu.VMEM_SHARED((NPAD, C), jnp.float32)]
        + [pltpu.SemaphoreType.DMA for _ in range(2 * _NBUF + _IR)]
    )
    if with_deg:
        out_type.append(jax.ShapeDtypeStruct((_NC, NPAD, _DEGW), jnp.float32))
        scratch += ([pltpu.VMEM_SHARED((NPAD, _DEGW), jnp.float32)]
                    + [pltpu.SemaphoreType.DMA for _ in range(_NBUF)])

    @functools.partial(
        pl.kernel,
        out_type=tuple(out_type),
        mesh=mesh,
        scratch_types=scratch,
        compiler_params=pltpu.CompilerParams(use_tc_tiling_on_sc=False),
    )
    def agg(x_hbm, cr_hbm, sum_hbm, *rest):
        if with_deg:
            deg_hbm = rest[0]
            rest = rest[1:]
        idx_r = rest[0]
        rowbufs = rest[1:1 + _NBUF]
        ones_v, zeros16_v, acc_sh = rest[1 + _NBUF:4 + _NBUF]
        gsems = rest[4 + _NBUF:4 + 2 * _NBUF]
        ssems = rest[4 + 2 * _NBUF:4 + 3 * _NBUF]
        isems = rest[4 + 3 * _NBUF:4 + 3 * _NBUF + _IR]
        if with_deg:
            deg_sh = rest[4 + 3 * _NBUF + _IR]
            dsems = rest[5 + 3 * _NBUF + _IR:5 + 4 * _NBUF + _IR]

        c = lax.axis_index("c")
        s = lax.axis_index("s")
        wbase = (c * _NS + s) * TW

        zeros = jnp.zeros((16,), jnp.float32)
        ones = jnp.ones((16,), jnp.float32)

        def memset_row(i, carry):
            for j in range(C // 16):
                rowbufs[0][i, pl.ds(16 * j, 16)] = zeros
            ones_v[i, pl.ds(0, 16)] = ones
            zeros16_v[i, pl.ds(0, 16)] = zeros
            return carry
        lax.fori_loop(0, _K, memset_row, 0)

        def zero_chunk(k, carry):
            g = k * _NS + s
            @pl.when(g < TZ)
            def _():
                r0 = g * _K
                pltpu.sync_copy(rowbufs[0], acc_sh.at[pl.ds(r0, _K)])
                if with_deg:
                    pltpu.sync_copy(zeros16_v, deg_sh.at[pl.ds(r0, _K)])
            return carry
        lax.fori_loop(0, ZPT, zero_chunk, 0)

        plsc.subcore_barrier()

        def prefetch(g, r):
            pltpu.async_copy(cr_hbm.at[wbase + g], idx_r.at[r], isems[r])

        def idx_wait(g, r):
            pltpu.make_async_copy(cr_hbm.at[wbase + g], idx_r.at[r],
                                  isems[r]).wait()

        def gather(r, b):
            pltpu.async_copy(x_hbm.at[idx_r.at[r, 0]], rowbufs[b], gsems[b])

        def gather_wait(r, b):
            pltpu.make_async_copy(x_hbm.at[idx_r.at[r, 0]], rowbufs[b],
                                  gsems[b]).wait()

        def scatter(r, b):
            pltpu.async_copy(rowbufs[b], acc_sh.at[idx_r.at[r, 1]],
                             ssems[b], add=True)
            if with_deg:
                pltpu.async_copy(ones_v, deg_sh.at[idx_r.at[r, 1]],
                                 dsems[b], add=True)

        def scatter_wait(r, b):
            pltpu.make_async_copy(rowbufs[b], acc_sh.at[idx_r.at[r, 1]],
                                  ssems[b]).wait()
            if with_deg:
                pltpu.make_async_copy(ones_v, deg_sh.at[idx_r.at[r, 1]],
                                      dsems[b]).wait()

        prefetch(0, 0)
        prefetch(1, 1)

        def pipe(u, carry):
            for r in range(_IR):
                g = u * _IR + r
                b = r % _NBUF

                @pl.when(g >= _NBUF)
                def _():
                    scatter_wait((r - _NBUF) % _IR, b)

                idx_wait(g, r)
                gather(r, b)

                @pl.when(g + 2 < TW)
                def _():
                    prefetch(g + 2, (r + 2) % _IR)

                @pl.when(g >= 1)
                def _():
                    gather_wait((r - 1) % _IR, 1 - b)
                    scatter((r - 1) % _IR, 1 - b)
            return carry
        lax.fori_loop(0, TW // _IR, pipe, 0)

        gather_wait((TW - 1) % _IR, (TW - 1) % _NBUF)
        scatter((TW - 1) % _IR, (TW - 1) % _NBUF)
        scatter_wait((TW - 2) % _IR, (TW - 2) % _NBUF)
        scatter_wait((TW - 1) % _IR, (TW - 1) % _NBUF)

        plsc.subcore_barrier()

        r0 = s * RPT
        pltpu.sync_copy(acc_sh.at[pl.ds(r0, RPT)], sum_hbm.at[c, pl.ds(r0, RPT)])
        if with_deg:
            pltpu.sync_copy(deg_sh.at[pl.ds(r0, RPT)],
                            deg_hbm.at[c, pl.ds(r0, RPT)])

    return agg


def _dense_layer(N, C, H, relu, BN=2000):
    def body(x_ref, p0_ref, p1_ref, d0_ref, d1_ref, w_ref, b_ref, o_ref):
        inv = 1.0 / jnp.maximum(d0_ref[0, :, 0:1] + d1_ref[0, :, 0:1], 1.0)
        agg = (p0_ref[0] + p1_ref[0]) * inv
        y = b_ref[...]
        y = y + lax.dot_general(x_ref[...], w_ref[:, :C],
                                (((1,), (1,)), ((), ())),
                                preferred_element_type=jnp.float32)
        y = y + lax.dot_general(agg, w_ref[:, C:],
                                (((1,), (1,)), ((), ())),
                                preferred_element_type=jnp.float32)
        if relu:
            y = jnp.maximum(y, 0.0)
        o_ref[...] = y

    return pl.pallas_call(
        body,
        grid=(N // BN,),
        in_specs=[
            pl.BlockSpec((BN, C), lambda i: (i, 0)),
            pl.BlockSpec((1, BN, C), lambda i: (0, i, 0)),
            pl.BlockSpec((1, BN, C), lambda i: (1, i, 0)),
            pl.BlockSpec((1, BN, _DEGW), lambda i: (0, i, 0)),
            pl.BlockSpec((1, BN, _DEGW), lambda i: (1, i, 0)),
            pl.BlockSpec((H, 2 * C), lambda i: (0, 0)),
            pl.BlockSpec((1, H), lambda i: (0, 0)),
        ],
        out_specs=pl.BlockSpec((BN, H), lambda i: (i, 0)),
        out_shape=jax.ShapeDtypeStruct((N, H), jnp.float32),
    )


def kernel(x, edge_index, W1, b1, W2, b2):
    N, C = x.shape
    H = W1.shape[0]
    O = W2.shape[0]
    E = edge_index.shape[1]
    align = _NW * _K * _IR
    EPAD = -(-E // align) * align

    rows = edge_index[0]
    cols = edge_index[1]
    pad = EPAD - E
    if pad:
        NPAD = -(-(N + 1) // _K) * _K
        arp = jnp.arange(pad, dtype=jnp.int32)
        rows = jnp.concatenate([rows, N + arp % (NPAD - N)])
        cols = jnp.concatenate([cols, arp % N])
    cr = jnp.stack([cols.reshape(EPAD // _K, _K),
                    rows.reshape(EPAD // _K, _K)], axis=1)

    sums, degs = _sc_aggregate(N, C, EPAD, True)(x, cr)
    h = _dense_layer(N, C, H, True)(
        x, sums, sums, degs, degs, W1, b1.reshape(1, H))

    (sums2,) = _sc_aggregate(N, H, EPAD, False)(h, cr)
    out = _dense_layer(N, H, O, False)(
        h, sums2, sums2, degs, degs, W2, b2.reshape(1, O))
    return out
